# baseline (speedup 1.0000x reference)
import functools

import jax
import jax.numpy as jnp
from jax.experimental import pallas as pl
from jax.experimental.pallas import tpu as pltpu

TOP_K = 8
N_EXPERTS = 64
N_GROUP = 8
GROUP_SIZE = N_EXPERTS // N_GROUP
TOPK_GROUP = 4
SCALE = 2.5

TBLK = 512
NEG = -1e30


def _router_body(h_ref, w_ref, b_ref, idx_ref, wgt_ref):
    x = h_ref[...]
    w = w_ref[...]
    logits = jax.lax.dot_general(
        x, w, (((1,), (1,)), ((), ())),
        preferred_element_type=jnp.float32,
        precision=jax.lax.Precision.DEFAULT,
    )
    scores = jax.nn.sigmoid(logits)
    s4c = scores + b_ref[...]

    t, e = scores.shape
    lane = jax.lax.broadcasted_iota(jnp.int32, (t, e), 1)
    grp = lane // GROUP_SIZE
    seg = lane % GROUP_SIZE

    a = s4c
    b = jnp.full((t, e), NEG, jnp.float32)
    for d in (4, 2, 1):
        wrap = seg >= (GROUP_SIZE - d)
        ra = jnp.where(wrap, jnp.roll(a, GROUP_SIZE - d, axis=1),
                       jnp.roll(a, -d, axis=1))
        rb = jnp.where(wrap, jnp.roll(b, GROUP_SIZE - d, axis=1),
                       jnp.roll(b, -d, axis=1))
        hi = jnp.maximum(a, ra)
        lo = jnp.minimum(a, ra)
        b = jnp.maximum(lo, jnp.maximum(b, rb))
        a = hi
    gs = a + b

    lane_f = lane.astype(jnp.float32)
    rev_f = (e - 1) - lane_f
    revgrp_f = jnp.floor(rev_f * (1.0 / GROUP_SIZE))

    cnt = jnp.zeros((t, e), jnp.float32)
    for k in range(1, N_GROUP):
        sgs = jnp.roll(gs, -GROUP_SIZE * k, axis=1)
        lower = ((grp + k) % N_GROUP) < grp
        beat = (sgs > gs) | ((sgs == gs) & lower)
        cnt = cnt + jnp.where(beat, 1.0, 0.0)
    sel = cnt < float(TOPK_GROUP)

    ms = jnp.where(sel, s4c, 0.0)
    avail2 = jnp.ones((t, e), jnp.bool_)
    idx_cols = []
    wgt_cols = []
    for _ in range(TOP_K):
        cur = jnp.where(avail2, ms, NEG)
        mv = jnp.max(cur, axis=1, keepdims=True)
        ilr = jnp.max(jnp.where(cur == mv, rev_f, NEG), axis=1, keepdims=True)
        avail2 = jnp.logical_and(avail2, rev_f != ilr)
        idx_cols.append(ilr)
        wgt_cols.append(mv)

    wsum = functools.reduce(jnp.add, wgt_cols)
    inv = 1.0 / (wsum + 1e-20)

    lane_k = jax.lax.broadcasted_iota(jnp.int32, (t, TOP_K), 1)
    idx_out = jnp.zeros((t, TOP_K), jnp.float32)
    wgt_out = jnp.zeros((t, TOP_K), jnp.float32)
    for k in range(TOP_K):
        sel_k = lane_k == k
        idx_out = jnp.where(sel_k, idx_cols[k], idx_out)
        wgt_out = jnp.where(sel_k, wgt_cols[k], wgt_out)
    wgt_out = (wgt_out * inv) * SCALE

    idx_ref[...] = ((e - 1) - idx_out).astype(jnp.int32)
    wgt_ref[...] = wgt_out


def kernel(hidden_states, weight, e_score_correction_bias):
    b, s, h = hidden_states.shape
    hf = hidden_states.reshape(-1, h).astype(jnp.float32)
    t = hf.shape[0]
    assert t % TBLK == 0
    bias2 = e_score_correction_bias.reshape(1, N_EXPERTS).astype(jnp.float32)
    grid = (t // TBLK,)
    idx, wgt = pl.pallas_call(
        _router_body,
        grid=grid,
        in_specs=[
            pl.BlockSpec((TBLK, h), lambda i: (i, 0)),
            pl.BlockSpec((N_EXPERTS, h), lambda i: (0, 0)),
            pl.BlockSpec((1, N_EXPERTS), lambda i: (0, 0)),
        ],
        out_specs=[
            pl.BlockSpec((TBLK, TOP_K), lambda i: (i, 0)),
            pl.BlockSpec((TBLK, TOP_K), lambda i: (i, 0)),
        ],
        out_shape=[
            jax.ShapeDtypeStruct((t, TOP_K), jnp.int32),
            jax.ShapeDtypeStruct((t, TOP_K), jnp.float32),
        ],
        compiler_params=pltpu.CompilerParams(
            dimension_semantics=("arbitrary",),
        ),
    )(hf, weight.astype(jnp.float32), bias2)
    return idx, wgt


# speedup vs baseline: 1.1101x; 1.1101x over previous
import functools

import jax
import jax.numpy as jnp
from jax.experimental import pallas as pl
from jax.experimental.pallas import tpu as pltpu

TOP_K = 8
N_EXPERTS = 64
N_GROUP = 8
GROUP_SIZE = N_EXPERTS // N_GROUP
TOPK_GROUP = 4
SCALE = 2.5

TBLK = 512
NEG = -1e30


def _router_body(h_ref, w_ref, b_ref, idx_ref, wgt_ref):
    x = h_ref[...]
    w = w_ref[...]
    logits = jax.lax.dot_general(
        x, w, (((1,), (1,)), ((), ())),
        preferred_element_type=jnp.float32,
        precision=jax.lax.Precision.DEFAULT,
    )
    scores = jax.nn.sigmoid(logits)
    s4c = scores + b_ref[...]

    t, e = scores.shape
    lane = jax.lax.broadcasted_iota(jnp.int32, (t, e), 1)
    grp = lane // GROUP_SIZE
    seg = lane % GROUP_SIZE

    def seg_roll(x, d):
        wrap = seg >= (GROUP_SIZE - d)
        return jnp.where(wrap, jnp.roll(x, GROUP_SIZE - d, axis=1),
                         jnp.roll(x, -d, axis=1))

    ra = seg_roll(s4c, 4)
    a = jnp.maximum(s4c, ra)
    b = jnp.minimum(s4c, ra)
    for d in (2, 1):
        ra = seg_roll(a, d)
        rb = seg_roll(b, d)
        hi = jnp.maximum(a, ra)
        lo = jnp.minimum(a, ra)
        b = jnp.maximum(lo, jnp.maximum(b, rb))
        a = hi
    gs = a + b

    lane_f = lane.astype(jnp.float32)
    rev_f = (e - 1) - lane_f
    revgrp_f = jnp.floor(rev_f * (1.0 / GROUP_SIZE))

    avail = jnp.ones((t, e), jnp.bool_)
    sel = jnp.zeros((t, e), jnp.bool_)
    for _ in range(TOPK_GROUP):
        cur = jnp.where(avail, gs, NEG)
        mv = jnp.max(cur, axis=1, keepdims=True)
        ilr = jnp.max(jnp.where(cur == mv, rev_f, NEG), axis=1, keepdims=True)
        gsel = revgrp_f == jnp.floor(ilr * (1.0 / GROUP_SIZE))
        sel = jnp.logical_or(sel, gsel)
        avail = jnp.logical_and(avail, jnp.logical_not(gsel))

    ms = jnp.where(sel, s4c, 0.0)
    idx_cols = []
    wgt_cols = []
    for _ in range(TOP_K):
        mv = jnp.max(ms, axis=1, keepdims=True)
        ilr = jnp.max(jnp.where(ms == mv, rev_f, NEG), axis=1, keepdims=True)
        ms = jnp.where(rev_f == ilr, NEG, ms)
        idx_cols.append(ilr)
        wgt_cols.append(mv)

    wsum = functools.reduce(jnp.add, wgt_cols)
    inv = 1.0 / (wsum + 1e-20)

    lane_k = jax.lax.broadcasted_iota(jnp.int32, (t, TOP_K), 1)
    idx_out = jnp.zeros((t, TOP_K), jnp.float32)
    wgt_out = jnp.zeros((t, TOP_K), jnp.float32)
    for k in range(TOP_K):
        sel_k = lane_k == k
        idx_out = jnp.where(sel_k, idx_cols[k], idx_out)
        wgt_out = jnp.where(sel_k, wgt_cols[k], wgt_out)
    wgt_out = (wgt_out * inv) * SCALE

    idx_ref[...] = ((e - 1) - idx_out).astype(jnp.int32)
    wgt_ref[...] = wgt_out


def kernel(hidden_states, weight, e_score_correction_bias):
    b, s, h = hidden_states.shape
    hf = hidden_states.reshape(-1, h).astype(jnp.float32)
    t = hf.shape[0]
    assert t % TBLK == 0
    bias2 = e_score_correction_bias.reshape(1, N_EXPERTS).astype(jnp.float32)
    grid = (t // TBLK,)
    idx, wgt = pl.pallas_call(
        _router_body,
        grid=grid,
        in_specs=[
            pl.BlockSpec((TBLK, h), lambda i: (i, 0)),
            pl.BlockSpec((N_EXPERTS, h), lambda i: (0, 0)),
            pl.BlockSpec((1, N_EXPERTS), lambda i: (0, 0)),
        ],
        out_specs=[
            pl.BlockSpec((TBLK, TOP_K), lambda i: (i, 0)),
            pl.BlockSpec((TBLK, TOP_K), lambda i: (i, 0)),
        ],
        out_shape=[
            jax.ShapeDtypeStruct((t, TOP_K), jnp.int32),
            jax.ShapeDtypeStruct((t, TOP_K), jnp.float32),
        ],
        compiler_params=pltpu.CompilerParams(
            dimension_semantics=("arbitrary",),
        ),
    )(hf, weight.astype(jnp.float32), bias2)
    return idx, wgt


# speedup vs baseline: 1.2056x; 1.0859x over previous
import functools

import jax
import jax.numpy as jnp
from jax.experimental import pallas as pl
from jax.experimental.pallas import tpu as pltpu

TOP_K = 8
N_EXPERTS = 64
N_GROUP = 8
GROUP_SIZE = N_EXPERTS // N_GROUP
TOPK_GROUP = 4
SCALE = 2.5

TBLK = 1024
NEG = -1e30


def _router_body(h_ref, w_ref, b_ref, idx_ref, wgt_ref):
    x = h_ref[...]
    w = w_ref[...]
    logits = jax.lax.dot_general(
        x, w, (((1,), (1,)), ((), ())),
        preferred_element_type=jnp.float32,
        precision=jax.lax.Precision.DEFAULT,
    )
    scores = jax.nn.sigmoid(logits)
    s4c = scores + b_ref[...]

    t, e = scores.shape
    lane = jax.lax.broadcasted_iota(jnp.int32, (t, e), 1)
    grp = lane // GROUP_SIZE
    seg = lane % GROUP_SIZE

    def seg_roll(x, d):
        wrap = seg >= (GROUP_SIZE - d)
        return jnp.where(wrap, jnp.roll(x, GROUP_SIZE - d, axis=1),
                         jnp.roll(x, -d, axis=1))

    ra = seg_roll(s4c, 4)
    a = jnp.maximum(s4c, ra)
    b = jnp.minimum(s4c, ra)
    for d in (2, 1):
        ra = seg_roll(a, d)
        rb = seg_roll(b, d)
        hi = jnp.maximum(a, ra)
        lo = jnp.minimum(a, ra)
        b = jnp.maximum(lo, jnp.maximum(b, rb))
        a = hi
    gs = a + b

    lane_f = lane.astype(jnp.float32)
    rev_f = (e - 1) - lane_f
    revgrp_f = jnp.floor(rev_f * (1.0 / GROUP_SIZE))

    avail = jnp.ones((t, e), jnp.bool_)
    sel = jnp.zeros((t, e), jnp.bool_)
    for _ in range(TOPK_GROUP):
        cur = jnp.where(avail, gs, NEG)
        mv = jnp.max(cur, axis=1, keepdims=True)
        ilr = jnp.max(jnp.where(cur == mv, rev_f, NEG), axis=1, keepdims=True)
        gsel = revgrp_f == jnp.floor(ilr * (1.0 / GROUP_SIZE))
        sel = jnp.logical_or(sel, gsel)
        avail = jnp.logical_and(avail, jnp.logical_not(gsel))

    ms = jnp.where(sel, s4c, 0.0)
    idx_cols = []
    wgt_cols = []
    for _ in range(TOP_K):
        mv = jnp.max(ms, axis=1, keepdims=True)
        ilr = jnp.max(jnp.where(ms == mv, rev_f, NEG), axis=1, keepdims=True)
        ms = jnp.where(rev_f == ilr, NEG, ms)
        idx_cols.append(ilr)
        wgt_cols.append(mv)

    wsum = functools.reduce(jnp.add, wgt_cols)
    inv = 1.0 / (wsum + 1e-20)

    lane_k = jax.lax.broadcasted_iota(jnp.int32, (t, TOP_K), 1)
    idx_out = jnp.zeros((t, TOP_K), jnp.float32)
    wgt_out = jnp.zeros((t, TOP_K), jnp.float32)
    for k in range(TOP_K):
        sel_k = lane_k == k
        idx_out = jnp.where(sel_k, idx_cols[k], idx_out)
        wgt_out = jnp.where(sel_k, wgt_cols[k], wgt_out)
    wgt_out = (wgt_out * inv) * SCALE

    idx_ref[...] = ((e - 1) - idx_out).astype(jnp.int32)
    wgt_ref[...] = wgt_out


def kernel(hidden_states, weight, e_score_correction_bias):
    b, s, h = hidden_states.shape
    hf = hidden_states.reshape(-1, h).astype(jnp.float32)
    t = hf.shape[0]
    assert t % TBLK == 0
    bias2 = e_score_correction_bias.reshape(1, N_EXPERTS).astype(jnp.float32)
    grid = (t // TBLK,)
    idx, wgt = pl.pallas_call(
        _router_body,
        grid=grid,
        in_specs=[
            pl.BlockSpec((TBLK, h), lambda i: (i, 0)),
            pl.BlockSpec((N_EXPERTS, h), lambda i: (0, 0)),
            pl.BlockSpec((1, N_EXPERTS), lambda i: (0, 0)),
        ],
        out_specs=[
            pl.BlockSpec((TBLK, TOP_K), lambda i: (i, 0)),
            pl.BlockSpec((TBLK, TOP_K), lambda i: (i, 0)),
        ],
        out_shape=[
            jax.ShapeDtypeStruct((t, TOP_K), jnp.int32),
            jax.ShapeDtypeStruct((t, TOP_K), jnp.float32),
        ],
        compiler_params=pltpu.CompilerParams(
            dimension_semantics=("arbitrary",),
        ),
    )(hf, weight.astype(jnp.float32), bias2)
    return idx, wgt
